# baseline (speedup 1.0000x reference)
import functools

import jax
import jax.numpy as jnp
from jax import lax
from jax.experimental import pallas as pl
from jax.experimental.pallas import tpu as pltpu
from jax.experimental.pallas import tpu_sc as plsc

_PACK_BASE = 100003
_B_LO = 34467
_NC, _NS = 2, 16
_NW = _NC * _NS
_L = 16
_C = 128
_W = 64
_BLK = 8
_SGN = -0x80000000


def _u32(x):
    return plsc.bitcast(x, jnp.uint32)


def _i32(x):
    return plsc.bitcast(x, jnp.int32)


def _carry(x, y, s):
    return ((x & y) | ((x | y) & ~s)) >> 31


def _pack_limbs(a, b, c):
    a, b, c = _u32(a), _u32(b), _u32(c)
    t = a * _B_LO
    s = t + b
    a_sh = a << 16
    u_lo = a_sh + s
    u_hi = (a >> 16) + _carry(a_sh, s, u_lo)
    p1 = u_lo >> 16
    p0 = u_lo & 0xFFFF
    q = p1 * _B_LO + p0
    q_sh = q << 16
    r = p0 * _B_LO
    x1 = q_sh + r
    c2 = _carry(q_sh, r, x1)
    key_lo = x1 + c
    c3 = _carry(x1, c, key_lo)
    key_hi = u_hi * _PACK_BASE + p1 + (q >> 16) + c2 + c3
    return _i32(key_hi), _i32(key_lo) ^ _SGN


def _make_sc_search(n, rows, sample, steps_a, chunks):
    mesh = plsc.VectorSubcoreMesh(
        core_axis_name="c", subcore_axis_name="s",
        num_cores=_NC, num_subcores=_NS)

    @functools.partial(
        pl.kernel,
        out_type=jax.ShapeDtypeStruct((n,), jnp.int32),
        mesh=mesh,
        scratch_types=[
            pltpu.VMEM((sample,), jnp.int32),
            pltpu.VMEM((sample,), jnp.int32),
            pltpu.VMEM((6 * _C,), jnp.int32),
            pltpu.VMEM((2, _C), jnp.int32),
            pltpu.VMEM((2, _C), jnp.int32),
            pltpu.VMEM((2, _C), jnp.int32),
            pltpu.VMEM((2, _C, 2 * _W), jnp.int32),
            pltpu.VMEM((_C,), jnp.int32),
            pltpu.SemaphoreType.DMA,
            pltpu.SemaphoreType.DMA,
        ],
        compiler_params=pltpu.CompilerParams(needs_layout_passes=False),
    )
    def sc_search(q_hbm, fcomb_hbm, shi_hbm, slo_hbm, out_hbm,
                  smp_hi, smp_lo, qv, bktv, qhv, qlv, win, outv, sem0, sem1):
        wid = lax.axis_index("s") * _NC + lax.axis_index("c")
        pltpu.sync_copy(shi_hbm, smp_hi)
        pltpu.sync_copy(slo_hbm, smp_lo)
        tile_base = wid * jnp.int32(chunks * _C)
        sems = (sem0, sem1)
        ngrp = _C // _L
        last_base = jnp.int32(n - _C)

        def phase_a(base, buf):
            buf = jnp.int32(buf)
            pltpu.sync_copy(q_hbm.at[pl.ds(base * jnp.int32(6), 6 * _C)], qv)
            for blk in range(0, ngrp, _BLK):
                keys, pos = [], []
                for g in range(blk, blk + _BLK):
                    idx6 = (lax.iota(jnp.int32, _L) + (g * _L)) * 6
                    a16 = plsc.load_gather(qv, [idx6])
                    b16 = plsc.load_gather(qv, [idx6 + 2])
                    c16 = plsc.load_gather(qv, [idx6 + 4])
                    qh, ql = _pack_limbs(a16, b16, c16)
                    qhv[buf, pl.ds(g * _L, _L)] = qh
                    qlv[buf, pl.ds(g * _L, _L)] = ql
                    keys.append((qh, ql))
                    pos.append(jnp.zeros((_L,), jnp.int32))
                for sstep in range(steps_a - 1, -1, -1):
                    ts, ths, tls = [], [], []
                    for i in range(_BLK):
                        t = pos[i] + (1 << sstep)
                        ts.append(t)
                        ths.append(plsc.load_gather(smp_hi, [t - 1]))
                        tls.append(plsc.load_gather(smp_lo, [t - 1]))
                    for i in range(_BLK):
                        qh, ql = keys[i]
                        less = (ths[i] < qh) | ((ths[i] == qh) & (tls[i] < ql))
                        pos[i] = jnp.where(less, ts[i], pos[i])
                for i in range(_BLK):
                    bktv[buf, pl.ds((blk + i) * _L, _L)] = (
                        jnp.minimum(pos[i], rows - 1))

        def fire(buf, k):
            return pltpu.async_copy(
                fcomb_hbm.at[bktv.at[jnp.int32(buf)]],
                win.at[jnp.int32(buf)], sems[k % 2])

        def gwait(buf):
            pltpu.make_async_copy(
                fcomb_hbm.at[bktv.at[jnp.int32(buf)]],
                win.at[jnp.int32(buf)], sems[buf]).wait()

        def phase_b(base, buf):
            buf = jnp.int32(buf)
            for blk in range(0, ngrp, _BLK):
                keys, pos, rws = [], [], []
                for g in range(blk, blk + _BLK):
                    qh = qhv[buf, pl.ds(g * _L, _L)]
                    ql = qlv[buf, pl.ds(g * _L, _L)]
                    keys.append((qh, ql))
                    rws.append(lax.iota(jnp.int32, _L) + (g * _L))
                    pos.append(jnp.zeros((_L,), jnp.int32))
                wref = win.at[buf]
                for sstep in range(5, -1, -1):
                    ts, ths, tls = [], [], []
                    for i in range(_BLK):
                        t = pos[i] + (1 << sstep)
                        t2 = t * 2
                        ts.append(t)
                        ths.append(plsc.load_gather(wref, [rws[i], t2 - 1]))
                        tls.append(
                            plsc.load_gather(wref, [rws[i], t2 - 2]) ^ _SGN)
                    for i in range(_BLK):
                        qh, ql = keys[i]
                        less = (ths[i] < qh) | ((ths[i] == qh) & (tls[i] < ql))
                        pos[i] = jnp.where(less, ts[i], pos[i])
                for i in range(_BLK):
                    qh, ql = keys[i]
                    p2 = pos[i] * 2
                    fh = plsc.load_gather(wref, [rws[i], p2 + 1])
                    fl = plsc.load_gather(wref, [rws[i], p2]) ^ _SGN
                    hit = (fh == qh) & (fl == ql)
                    outv[pl.ds((blk + i) * _L, _L)] = hit.astype(jnp.int32)
            pltpu.sync_copy(outv, out_hbm.at[pl.ds(base, _C)])

        def cbase(k):
            return jnp.minimum(tile_base + k * jnp.int32(_C), last_base)

        phase_a(cbase(jnp.int32(0)), 0)
        fire(0, 0)

        def pair_body(j, carry):
            k = j * jnp.int32(2)
            phase_a(cbase(k + 1), 1)
            fire(1, 1)
            gwait(0)
            phase_b(cbase(k), 0)
            phase_a(cbase(k + 2), 0)
            fire(0, 0)
            gwait(1)
            phase_b(cbase(k + 1), 1)
            return carry

        lax.fori_loop(jnp.int32(0), jnp.int32((chunks - 1) // 2), pair_body,
                      jnp.int32(0))
        gwait(0)
        phase_b(cbase(jnp.int32(chunks - 1)), 0)

    return sc_search


@jax.jit
def _fact_index(atoms, fact_hashes):
    n = atoms.shape[0]
    f = fact_hashes.shape[0]

    if f % _W:
        pad = _W - f % _W
        fact_hashes = jnp.concatenate(
            [fact_hashes, jnp.full((pad,), (1 << 62), jnp.int64)])
        f += pad
    rows = f // _W
    fcomb = lax.bitcast_convert_type(
        fact_hashes, jnp.int32).reshape(rows, 2 * _W)
    sample = max(2, 1 << (rows - 1).bit_length())
    steps_a = sample.bit_length() - 1
    smp_hi = jnp.concatenate(
        [fcomb[:, 2 * _W - 1], jnp.full((sample - rows,), 0x7FFFFFFF,
                                        jnp.int32)])
    smp_lo = jnp.concatenate(
        [fcomb[:, 2 * _W - 2], jnp.full((sample - rows,), -1,
                                        jnp.int32)]) ^ jnp.int32(_SGN)

    qflat = lax.bitcast_convert_type(atoms, jnp.int32).reshape(-1)
    chunks = (n + _NW * _C - 1) // (_NW * _C)
    if chunks % 2 == 0:
        chunks += 1

    sc_search = _make_sc_search(n, rows, sample, steps_a, chunks)
    out = sc_search(qflat, fcomb, smp_hi, smp_lo)
    return out != 0


def kernel(atoms, fact_hashes):
    if atoms.shape[0] == 0 or fact_hashes.shape[0] == 0:
        return jnp.zeros((atoms.shape[0],), dtype=bool)
    return _fact_index(atoms, fact_hashes)


# speedup vs baseline: 4.1911x; 4.1911x over previous
import functools

import jax
import jax.numpy as jnp
from jax import lax
from jax.experimental import pallas as pl
from jax.experimental.pallas import tpu as pltpu
from jax.experimental.pallas import tpu_sc as plsc

_PACK_BASE = 100003
_B_LO = 34467
_NC, _NS = 2, 16
_NW = _NC * _NS
_L = 16
_C = 128
_W = 64
_BLK = 8
_SGN = -0x80000000


def _u32(x):
    return plsc.bitcast(x, jnp.uint32)


def _i32(x):
    return plsc.bitcast(x, jnp.int32)


def _carry(x, y, s):
    return ((x & y) | ((x | y) & ~s)) >> 31


def _pack_limbs(a, b, c):
    a, b, c = _u32(a), _u32(b), _u32(c)
    t = a * _B_LO
    s = t + b
    a_sh = a << 16
    u_lo = a_sh + s
    u_hi = (a >> 16) + _carry(a_sh, s, u_lo)
    p1 = u_lo >> 16
    p0 = u_lo & 0xFFFF
    q = p1 * _B_LO + p0
    q_sh = q << 16
    r = p0 * _B_LO
    x1 = q_sh + r
    c2 = _carry(q_sh, r, x1)
    key_lo = x1 + c
    c3 = _carry(x1, c, key_lo)
    key_hi = u_hi * _PACK_BASE + p1 + (q >> 16) + c2 + c3
    return _i32(key_hi), _i32(key_lo) ^ _SGN


def _make_sc_search(n, rows, sample, steps_a, chunks):
    mesh = plsc.VectorSubcoreMesh(
        core_axis_name="c", subcore_axis_name="s",
        num_cores=_NC, num_subcores=_NS)

    @functools.partial(
        pl.kernel,
        out_type=jax.ShapeDtypeStruct((n,), jnp.int32),
        mesh=mesh,
        scratch_types=[
            pltpu.VMEM((2 * sample,), jnp.int32),
            pltpu.VMEM((_C,), jnp.int32),
            pltpu.VMEM((_C,), jnp.int32),
            pltpu.VMEM((_C,), jnp.int32),
            pltpu.VMEM((2, _C), jnp.int32),
            pltpu.VMEM((2, _C), jnp.int32),
            pltpu.VMEM((2, _C), jnp.int32),
            pltpu.VMEM((2, _C, 2 * _W), jnp.int32),
            pltpu.VMEM((_C,), jnp.int32),
            pltpu.SemaphoreType.DMA,
            pltpu.SemaphoreType.DMA,
        ],
        compiler_params=pltpu.CompilerParams(needs_layout_passes=False),
    )
    def sc_search(a_hbm, b_hbm, c_hbm, fcomb_hbm, smp_hbm, out_hbm,
                  smp, av, bv, cv, bktv, qhv, qlv, win, outv, sem0, sem1):
        wid = lax.axis_index("s") * _NC + lax.axis_index("c")
        pltpu.sync_copy(smp_hbm, smp)
        tile_base = wid * jnp.int32(chunks * _C)
        sems = (sem0, sem1)
        ngrp = _C // _L
        last_base = jnp.int32(n - _C)

        def phase_a(base, buf):
            buf = jnp.int32(buf)
            pltpu.sync_copy(a_hbm.at[pl.ds(base, _C)], av)
            pltpu.sync_copy(b_hbm.at[pl.ds(base, _C)], bv)
            pltpu.sync_copy(c_hbm.at[pl.ds(base, _C)], cv)
            for blk in range(0, ngrp, _BLK):
                keys, pos = [], []
                for g in range(blk, blk + _BLK):
                    a16 = av[pl.ds(g * _L, _L)]
                    b16 = bv[pl.ds(g * _L, _L)]
                    c16 = cv[pl.ds(g * _L, _L)]
                    qh, ql = _pack_limbs(a16, b16, c16)
                    qhv[buf, pl.ds(g * _L, _L)] = qh
                    qlv[buf, pl.ds(g * _L, _L)] = ql
                    keys.append((qh, ql))
                    pos.append(jnp.zeros((_L,), jnp.int32))
                for sstep in range(steps_a - 1, -1, -1):
                    ts, ths, tls = [], [], []
                    for i in range(_BLK):
                        t = pos[i] + (1 << sstep)
                        t2 = t * 2
                        ts.append(t)
                        ths.append(plsc.load_gather(smp, [t2 - 1]))
                        tls.append(plsc.load_gather(smp, [t2 - 2]) ^ _SGN)
                    for i in range(_BLK):
                        qh, ql = keys[i]
                        less = (ths[i] < qh) | ((ths[i] == qh) & (tls[i] < ql))
                        pos[i] = jnp.where(less, ts[i], pos[i])
                for i in range(_BLK):
                    bktv[buf, pl.ds((blk + i) * _L, _L)] = (
                        jnp.minimum(pos[i], rows - 1))

        def fire(buf, k):
            return pltpu.async_copy(
                fcomb_hbm.at[bktv.at[jnp.int32(buf)]],
                win.at[jnp.int32(buf)], sems[k % 2])

        def gwait(buf):
            pltpu.make_async_copy(
                fcomb_hbm.at[bktv.at[jnp.int32(buf)]],
                win.at[jnp.int32(buf)], sems[buf]).wait()

        def phase_b(base, buf):
            buf = jnp.int32(buf)
            for blk in range(0, ngrp, _BLK):
                keys, pos, rws = [], [], []
                for g in range(blk, blk + _BLK):
                    qh = qhv[buf, pl.ds(g * _L, _L)]
                    ql = qlv[buf, pl.ds(g * _L, _L)]
                    keys.append((qh, ql))
                    rws.append(lax.iota(jnp.int32, _L) + (g * _L))
                    pos.append(jnp.zeros((_L,), jnp.int32))
                wref = win.at[buf]
                for sstep in range(5, -1, -1):
                    ts, ths, tls = [], [], []
                    for i in range(_BLK):
                        t = pos[i] + (1 << sstep)
                        t2 = t * 2
                        ts.append(t)
                        ths.append(plsc.load_gather(wref, [rws[i], t2 - 1]))
                        tls.append(
                            plsc.load_gather(wref, [rws[i], t2 - 2]) ^ _SGN)
                    for i in range(_BLK):
                        qh, ql = keys[i]
                        less = (ths[i] < qh) | ((ths[i] == qh) & (tls[i] < ql))
                        pos[i] = jnp.where(less, ts[i], pos[i])
                for i in range(_BLK):
                    qh, ql = keys[i]
                    p2 = pos[i] * 2
                    fh = plsc.load_gather(wref, [rws[i], p2 + 1])
                    fl = plsc.load_gather(wref, [rws[i], p2]) ^ _SGN
                    hit = (fh == qh) & (fl == ql)
                    outv[pl.ds((blk + i) * _L, _L)] = hit.astype(jnp.int32)
            pltpu.sync_copy(outv, out_hbm.at[pl.ds(base, _C)])

        def cbase(k):
            return jnp.minimum(tile_base + k * jnp.int32(_C), last_base)

        phase_a(cbase(jnp.int32(0)), 0)
        fire(0, 0)

        def pair_body(j, carry):
            k = j * jnp.int32(2)
            phase_a(cbase(k + 1), 1)
            fire(1, 1)
            gwait(0)
            phase_b(cbase(k), 0)
            phase_a(cbase(k + 2), 0)
            fire(0, 0)
            gwait(1)
            phase_b(cbase(k + 1), 1)
            return carry

        lax.fori_loop(jnp.int32(0), jnp.int32((chunks - 1) // 2), pair_body,
                      jnp.int32(0))
        gwait(0)
        phase_b(cbase(jnp.int32(chunks - 1)), 0)

    return sc_search


@jax.jit
def _fact_index(atoms, fact_hashes):
    n = atoms.shape[0]
    f = fact_hashes.shape[0]

    if f % _W:
        pad = _W - f % _W
        fact_hashes = jnp.concatenate(
            [fact_hashes, jnp.full((pad,), (1 << 62), jnp.int64)])
        f += pad
    rows = f // _W
    fcomb = lax.bitcast_convert_type(
        fact_hashes, jnp.int32).reshape(rows, 2 * _W)
    sample = max(2, 1 << (rows - 1).bit_length())
    steps_a = sample.bit_length() - 1
    smp64 = jnp.max(fact_hashes.reshape(rows, _W), axis=1)
    smp64 = jnp.concatenate(
        [smp64, jnp.full((sample - rows,), (1 << 62) - 1, jnp.int64)])
    smp_flat = lax.bitcast_convert_type(smp64, jnp.int32).reshape(-1)

    cols = [atoms[:, j].astype(jnp.int32) for j in range(3)]
    chunks = (n + _NW * _C - 1) // (_NW * _C)
    if chunks % 2 == 0:
        chunks += 1

    sc_search = _make_sc_search(n, rows, sample, steps_a, chunks)
    out = sc_search(cols[0], cols[1], cols[2], fcomb, smp_flat)
    return out != 0


def kernel(atoms, fact_hashes):
    if atoms.shape[0] == 0 or fact_hashes.shape[0] == 0:
        return jnp.zeros((atoms.shape[0],), dtype=bool)
    return _fact_index(atoms, fact_hashes)


# speedup vs baseline: 4.2108x; 1.0047x over previous
import functools

import jax
import jax.numpy as jnp
from jax import lax
from jax.experimental import pallas as pl
from jax.experimental.pallas import tpu as pltpu
from jax.experimental.pallas import tpu_sc as plsc

_PACK_BASE = 100003
_B_LO = 34467
_NC, _NS = 2, 16
_NW = _NC * _NS
_L = 16
_C = 128
_W = 64
_BLK = 8
_SGN = -0x80000000


def _u32(x):
    return plsc.bitcast(x, jnp.uint32)


def _i32(x):
    return plsc.bitcast(x, jnp.int32)


def _carry(x, y, s):
    return ((x & y) | ((x | y) & ~s)) >> 31


def _pack_limbs(a, b, c):
    a, b, c = _u32(a), _u32(b), _u32(c)
    t = a * _B_LO
    s = t + b
    a_sh = a << 16
    u_lo = a_sh + s
    u_hi = (a >> 16) + _carry(a_sh, s, u_lo)
    p1 = u_lo >> 16
    p0 = u_lo & 0xFFFF
    q = p1 * _B_LO + p0
    q_sh = q << 16
    r = p0 * _B_LO
    x1 = q_sh + r
    c2 = _carry(q_sh, r, x1)
    key_lo = x1 + c
    c3 = _carry(x1, c, key_lo)
    key_hi = u_hi * _PACK_BASE + p1 + (q >> 16) + c2 + c3
    return _i32(key_hi), _i32(key_lo) ^ _SGN


def _make_sc_search(n, rows, sample, steps_a, chunks):
    mesh = plsc.VectorSubcoreMesh(
        core_axis_name="c", subcore_axis_name="s",
        num_cores=_NC, num_subcores=_NS)

    @functools.partial(
        pl.kernel,
        out_type=jax.ShapeDtypeStruct((n,), jnp.int32),
        mesh=mesh,
        scratch_types=[
            pltpu.VMEM((2 * sample,), jnp.int32),
            pltpu.VMEM((_C,), jnp.int32),
            pltpu.VMEM((_C,), jnp.int32),
            pltpu.VMEM((_C,), jnp.int32),
            pltpu.VMEM((2, _C), jnp.int32),
            pltpu.VMEM((2, _C), jnp.int32),
            pltpu.VMEM((2, _C), jnp.int32),
            pltpu.VMEM((2, _C, 2 * _W), jnp.int32),
            pltpu.VMEM((_C,), jnp.int32),
            pltpu.SemaphoreType.DMA,
            pltpu.SemaphoreType.DMA,
        ],
        compiler_params=pltpu.CompilerParams(needs_layout_passes=False),
    )
    def sc_search(a_hbm, b_hbm, c_hbm, fcomb_hbm, smp_hbm, out_hbm,
                  smp, av, bv, cv, bktv, qhv, qlv, win, outv, sem0, sem1):
        wid = lax.axis_index("s") * _NC + lax.axis_index("c")
        pltpu.sync_copy(smp_hbm, smp)
        tile_base = wid * jnp.int32(chunks * _C)
        sems = (sem0, sem1)
        ngrp = _C // _L
        last_base = jnp.int32(n - _C)

        def phase_a(base, buf):
            buf = jnp.int32(buf)
            pltpu.sync_copy(a_hbm.at[pl.ds(base, _C)], av)
            pltpu.sync_copy(b_hbm.at[pl.ds(base, _C)], bv)
            pltpu.sync_copy(c_hbm.at[pl.ds(base, _C)], cv)
            for blk in range(0, ngrp, _BLK):
                keys, pos = [], []
                for g in range(blk, blk + _BLK):
                    a16 = av[pl.ds(g * _L, _L)]
                    b16 = bv[pl.ds(g * _L, _L)]
                    c16 = cv[pl.ds(g * _L, _L)]
                    qh, ql = _pack_limbs(a16, b16, c16)
                    qhv[buf, pl.ds(g * _L, _L)] = qh
                    qlv[buf, pl.ds(g * _L, _L)] = ql
                    keys.append((qh, ql))
                    pos.append(jnp.zeros((_L,), jnp.int32))
                for sstep in range(steps_a - 1, -1, -1):
                    ts, ths, tls = [], [], []
                    for i in range(_BLK):
                        t = pos[i] + (1 << sstep)
                        t2 = t * 2
                        ts.append(t)
                        ths.append(plsc.load_gather(smp, [t2 - 1]))
                        tls.append(plsc.load_gather(smp, [t2 - 2]) ^ _SGN)
                    for i in range(_BLK):
                        qh, ql = keys[i]
                        less = (ths[i] < qh) | ((ths[i] == qh) & (tls[i] < ql))
                        pos[i] = jnp.where(less, ts[i], pos[i])
                for i in range(_BLK):
                    bktv[buf, pl.ds((blk + i) * _L, _L)] = (
                        jnp.minimum(pos[i], rows - 1))

        def fire(buf, k):
            return pltpu.async_copy(
                fcomb_hbm.at[bktv.at[jnp.int32(buf)]],
                win.at[jnp.int32(buf)], sems[k % 2])

        def gwait(buf):
            pltpu.make_async_copy(
                fcomb_hbm.at[bktv.at[jnp.int32(buf)]],
                win.at[jnp.int32(buf)], sems[buf]).wait()

        def phase_b(base, buf):
            buf = jnp.int32(buf)
            for blk in range(0, ngrp, _BLK):
                keys, pos, rws = [], [], []
                for g in range(blk, blk + _BLK):
                    qh = qhv[buf, pl.ds(g * _L, _L)]
                    ql = qlv[buf, pl.ds(g * _L, _L)]
                    keys.append((qh, ql))
                    rws.append(lax.iota(jnp.int32, _L) + (g * _L))
                    pos.append(jnp.zeros((_L,), jnp.int32))
                wref = win.at[buf]
                for sstep in range(5, -1, -1):
                    ts, ths, tls = [], [], []
                    for i in range(_BLK):
                        t = pos[i] + (1 << sstep)
                        t2 = t * 2
                        ts.append(t)
                        ths.append(plsc.load_gather(wref, [rws[i], t2 - 1]))
                        tls.append(
                            plsc.load_gather(wref, [rws[i], t2 - 2]) ^ _SGN)
                    for i in range(_BLK):
                        qh, ql = keys[i]
                        less = (ths[i] < qh) | ((ths[i] == qh) & (tls[i] < ql))
                        pos[i] = jnp.where(less, ts[i], pos[i])
                for i in range(_BLK):
                    qh, ql = keys[i]
                    p2 = pos[i] * 2
                    fh = plsc.load_gather(wref, [rws[i], p2 + 1])
                    fl = plsc.load_gather(wref, [rws[i], p2]) ^ _SGN
                    hit = (fh == qh) & (fl == ql)
                    outv[pl.ds((blk + i) * _L, _L)] = hit.astype(jnp.int32)
            pltpu.sync_copy(outv, out_hbm.at[pl.ds(base, _C)])

        def cbase(k):
            return jnp.minimum(tile_base + k * jnp.int32(_C), last_base)

        phase_a(cbase(jnp.int32(0)), 0)
        fire(0, 0)

        def pair_body(j, carry):
            k = j * jnp.int32(2)
            phase_a(cbase(k + 1), 1)
            fire(1, 1)
            gwait(0)
            phase_b(cbase(k), 0)
            phase_a(cbase(k + 2), 0)
            fire(0, 0)
            gwait(1)
            phase_b(cbase(k + 1), 1)
            return carry

        lax.fori_loop(jnp.int32(0), jnp.int32((chunks - 1) // 2), pair_body,
                      jnp.int32(0))
        gwait(0)
        phase_b(cbase(jnp.int32(chunks - 1)), 0)

    return sc_search


@jax.jit
def _fact_index(atoms, fact_hashes):
    n = atoms.shape[0]
    f = fact_hashes.shape[0]

    if f % _W:
        pad = _W - f % _W
        fact_hashes = jnp.concatenate(
            [fact_hashes, jnp.full((pad,), (1 << 62), jnp.int64)])
        f += pad
    rows = f // _W
    fhi = (fact_hashes >> 32).astype(jnp.int32)
    flo = fact_hashes.astype(jnp.int32)
    fcomb = jnp.stack(
        [flo.reshape(rows, _W), fhi.reshape(rows, _W)],
        axis=2).reshape(rows, 2 * _W)
    sample = max(2, 1 << (rows - 1).bit_length())
    steps_a = sample.bit_length() - 1
    smp64 = jnp.max(fact_hashes.reshape(rows, _W), axis=1)
    smp_hi = jnp.concatenate(
        [(smp64 >> 32).astype(jnp.int32),
         jnp.full((sample - rows,), 0x7FFFFFFF, jnp.int32)])
    smp_lo = jnp.concatenate(
        [smp64.astype(jnp.int32), jnp.full((sample - rows,), -1, jnp.int32)])
    smp_flat = jnp.stack([smp_lo, smp_hi], axis=1).reshape(-1)

    cols = [atoms[:, j].astype(jnp.int32) for j in range(3)]
    chunks = (n + _NW * _C - 1) // (_NW * _C)
    if chunks % 2 == 0:
        chunks += 1

    sc_search = _make_sc_search(n, rows, sample, steps_a, chunks)
    out = sc_search(cols[0], cols[1], cols[2], fcomb, smp_flat)
    return out != 0


def kernel(atoms, fact_hashes):
    if atoms.shape[0] == 0 or fact_hashes.shape[0] == 0:
        return jnp.zeros((atoms.shape[0],), dtype=bool)
    return _fact_index(atoms, fact_hashes)


# speedup vs baseline: 5.4648x; 1.2978x over previous
import functools

import jax
import jax.numpy as jnp
from jax import lax
from jax.experimental import pallas as pl
from jax.experimental.pallas import tpu as pltpu
from jax.experimental.pallas import tpu_sc as plsc

_PACK_BASE = 100003
_B_LO = 34467
_NC, _NS = 2, 16
_NW = _NC * _NS
_L = 16
_C = 128
_W = 64
_BLK = 8
_SGN = -0x80000000


def _u32(x):
    return plsc.bitcast(x, jnp.uint32)


def _i32(x):
    return plsc.bitcast(x, jnp.int32)


def _carry(x, y, s):
    return ((x & y) | ((x | y) & ~s)) >> 31


def _pack_limbs(a, b, c):
    a, b, c = _u32(a), _u32(b), _u32(c)
    t = a * _B_LO
    s = t + b
    a_sh = a << 16
    u_lo = a_sh + s
    u_hi = (a >> 16) + _carry(a_sh, s, u_lo)
    p1 = u_lo >> 16
    p0 = u_lo & 0xFFFF
    q = p1 * _B_LO + p0
    q_sh = q << 16
    r = p0 * _B_LO
    x1 = q_sh + r
    c2 = _carry(q_sh, r, x1)
    key_lo = x1 + c
    c3 = _carry(x1, c, key_lo)
    key_hi = u_hi * _PACK_BASE + p1 + (q >> 16) + c2 + c3
    return _i32(key_hi), _i32(key_lo) ^ _SGN


def _make_sc_search(n, rows, sample, steps_a, chunks):
    mesh = plsc.VectorSubcoreMesh(
        core_axis_name="c", subcore_axis_name="s",
        num_cores=_NC, num_subcores=_NS)

    @functools.partial(
        pl.kernel,
        out_type=jax.ShapeDtypeStruct((n,), jnp.int32),
        mesh=mesh,
        scratch_types=[
            pltpu.VMEM((2 * sample,), jnp.int32),
            pltpu.VMEM((2, _C), jnp.int32),
            pltpu.VMEM((2, _C), jnp.int32),
            pltpu.VMEM((2, _C), jnp.int32),
            pltpu.VMEM((2, _C), jnp.int32),
            pltpu.VMEM((2, _C), jnp.int32),
            pltpu.VMEM((2, _C), jnp.int32),
            pltpu.VMEM((2, _C, 2 * _W), jnp.int32),
            pltpu.VMEM((2, _C), jnp.int32),
            pltpu.SemaphoreType.DMA,
            pltpu.SemaphoreType.DMA,
            pltpu.SemaphoreType.DMA,
            pltpu.SemaphoreType.DMA,
            pltpu.SemaphoreType.DMA,
            pltpu.SemaphoreType.DMA,
        ],
        compiler_params=pltpu.CompilerParams(needs_layout_passes=False),
    )
    def sc_search(a_hbm, b_hbm, c_hbm, fcomb_hbm, smp_hbm, out_hbm,
                  smp, av, bv, cv, bktv, qhv, qlv, win, outv,
                  gsem0, gsem1, qsem0, qsem1, osem0, osem1):
        wid = lax.axis_index("s") * _NC + lax.axis_index("c")
        pltpu.sync_copy(smp_hbm, smp)
        tile_base = wid * jnp.int32(chunks * _C)
        gsems = (gsem0, gsem1)
        qsems = (qsem0, qsem1)
        osems = (osem0, osem1)
        ngrp = _C // _L
        last_base = jnp.int32(n - _C)

        def cbase(k):
            k = jnp.maximum(k, jnp.int32(0))
            return jnp.minimum(tile_base + k * jnp.int32(_C), last_base)

        def q_copies(base, buf):
            b = jnp.int32(buf)
            sl = pl.ds(base, _C)
            return [(a_hbm.at[sl], av.at[b]), (b_hbm.at[sl], bv.at[b]),
                    (c_hbm.at[sl], cv.at[b])]

        def qfire(base, buf):
            for src, dst in q_copies(base, buf):
                pltpu.async_copy(src, dst, qsems[buf])

        def qwait(base, buf):
            for src, dst in q_copies(base, buf):
                pltpu.make_async_copy(src, dst, qsems[buf]).wait()

        def phase_a(buf):
            buf = jnp.int32(buf)
            for blk in range(0, ngrp, _BLK):
                keys, pos = [], []
                for g in range(blk, blk + _BLK):
                    a16 = av[buf, pl.ds(g * _L, _L)]
                    b16 = bv[buf, pl.ds(g * _L, _L)]
                    c16 = cv[buf, pl.ds(g * _L, _L)]
                    qh, ql = _pack_limbs(a16, b16, c16)
                    qhv[buf, pl.ds(g * _L, _L)] = qh
                    qlv[buf, pl.ds(g * _L, _L)] = ql
                    keys.append((qh, ql))
                    pos.append(jnp.zeros((_L,), jnp.int32))
                for sstep in range(steps_a - 1, -1, -1):
                    ts, ths, tls = [], [], []
                    for i in range(_BLK):
                        t = pos[i] + (1 << sstep)
                        t2 = t * 2
                        ts.append(t)
                        ths.append(plsc.load_gather(smp, [t2 - 1]))
                        tls.append(plsc.load_gather(smp, [t2 - 2]) ^ _SGN)
                    for i in range(_BLK):
                        qh, ql = keys[i]
                        less = (ths[i] < qh) | ((ths[i] == qh) & (tls[i] < ql))
                        pos[i] = jnp.where(less, ts[i], pos[i])
                for i in range(_BLK):
                    bktv[buf, pl.ds((blk + i) * _L, _L)] = (
                        jnp.minimum(pos[i], rows - 1))

        def gfire(buf):
            pltpu.async_copy(
                fcomb_hbm.at[bktv.at[jnp.int32(buf)]],
                win.at[jnp.int32(buf)], gsems[buf])

        def gwait(buf):
            pltpu.make_async_copy(
                fcomb_hbm.at[bktv.at[jnp.int32(buf)]],
                win.at[jnp.int32(buf)], gsems[buf]).wait()

        def ofire(base, buf):
            pltpu.async_copy(
                outv.at[jnp.int32(buf)], out_hbm.at[pl.ds(base, _C)],
                osems[buf])

        def owait(base, buf):
            pltpu.make_async_copy(
                outv.at[jnp.int32(buf)], out_hbm.at[pl.ds(base, _C)],
                osems[buf]).wait()

        def phase_b(buf):
            buf = jnp.int32(buf)
            for blk in range(0, ngrp, _BLK):
                keys, pos, rws = [], [], []
                for g in range(blk, blk + _BLK):
                    qh = qhv[buf, pl.ds(g * _L, _L)]
                    ql = qlv[buf, pl.ds(g * _L, _L)]
                    keys.append((qh, ql))
                    rws.append(lax.iota(jnp.int32, _L) + (g * _L))
                    pos.append(jnp.zeros((_L,), jnp.int32))
                wref = win.at[buf]
                for sstep in range(5, -1, -1):
                    ts, ths, tls = [], [], []
                    for i in range(_BLK):
                        t = pos[i] + (1 << sstep)
                        t2 = t * 2
                        ts.append(t)
                        ths.append(plsc.load_gather(wref, [rws[i], t2 - 1]))
                        tls.append(
                            plsc.load_gather(wref, [rws[i], t2 - 2]) ^ _SGN)
                    for i in range(_BLK):
                        qh, ql = keys[i]
                        less = (ths[i] < qh) | ((ths[i] == qh) & (tls[i] < ql))
                        pos[i] = jnp.where(less, ts[i], pos[i])
                for i in range(_BLK):
                    qh, ql = keys[i]
                    p2 = pos[i] * 2
                    fh = plsc.load_gather(wref, [rws[i], p2 + 1])
                    fl = plsc.load_gather(wref, [rws[i], p2]) ^ _SGN
                    hit = (fh == qh) & (fl == ql)
                    outv[buf, pl.ds((blk + i) * _L, _L)] = (
                        hit.astype(jnp.int32))

        pltpu.async_copy(out_hbm.at[pl.ds(cbase(jnp.int32(0)), _C)],
                         outv.at[jnp.int32(0)], osem0)
        pltpu.async_copy(out_hbm.at[pl.ds(cbase(jnp.int32(0)), _C)],
                         outv.at[jnp.int32(1)], osem1)
        qfire(cbase(jnp.int32(0)), 0)
        qwait(cbase(jnp.int32(0)), 0)
        qfire(cbase(jnp.int32(1)), 1)
        phase_a(0)
        gfire(0)

        def pair_body(j, carry):
            k = j * jnp.int32(2)
            qwait(cbase(k + 1), 1)
            qfire(cbase(k + 2), 0)
            phase_a(1)
            gfire(1)
            gwait(0)
            owait(cbase(k - 2), 0)
            phase_b(0)
            ofire(cbase(k), 0)

            qwait(cbase(k + 2), 0)
            qfire(cbase(k + 3), 1)
            phase_a(0)
            gfire(0)
            gwait(1)
            owait(cbase(k - 1), 1)
            phase_b(1)
            ofire(cbase(k + 1), 1)
            return carry

        nloop = jnp.int32((chunks - 1) // 2)
        lax.fori_loop(jnp.int32(0), nloop, pair_body, jnp.int32(0))
        lastk = jnp.int32(chunks - 1)
        qwait(cbase(lastk + 1), 1)
        gwait(0)
        owait(cbase(lastk - 2), 0)
        phase_b(0)
        ofire(cbase(lastk), 0)
        owait(cbase(lastk - 1), 1)
        owait(cbase(lastk), 0)

    return sc_search


@jax.jit
def _fact_index(atoms, fact_hashes):
    n = atoms.shape[0]
    f = fact_hashes.shape[0]

    if f % _W:
        pad = _W - f % _W
        fact_hashes = jnp.concatenate(
            [fact_hashes, jnp.full((pad,), (1 << 62), jnp.int64)])
        f += pad
    rows = f // _W
    fhi = (fact_hashes >> 32).astype(jnp.int32)
    flo = fact_hashes.astype(jnp.int32)
    fcomb = jnp.stack(
        [flo.reshape(rows, _W), fhi.reshape(rows, _W)],
        axis=2).reshape(rows, 2 * _W)
    sample = max(2, 1 << (rows - 1).bit_length())
    steps_a = sample.bit_length() - 1
    smp64 = jnp.max(fact_hashes.reshape(rows, _W), axis=1)
    smp_hi = jnp.concatenate(
        [(smp64 >> 32).astype(jnp.int32),
         jnp.full((sample - rows,), 0x7FFFFFFF, jnp.int32)])
    smp_lo = jnp.concatenate(
        [smp64.astype(jnp.int32), jnp.full((sample - rows,), -1, jnp.int32)])
    smp_flat = jnp.stack([smp_lo, smp_hi], axis=1).reshape(-1)

    cols = [atoms[:, j].astype(jnp.int32) for j in range(3)]
    chunks = (n + _NW * _C - 1) // (_NW * _C)
    if chunks % 2 == 0:
        chunks += 1

    sc_search = _make_sc_search(n, rows, sample, steps_a, chunks)
    out = sc_search(cols[0], cols[1], cols[2], fcomb, smp_flat)
    return out != 0


def kernel(atoms, fact_hashes):
    if atoms.shape[0] == 0 or fact_hashes.shape[0] == 0:
        return jnp.zeros((atoms.shape[0],), dtype=bool)
    return _fact_index(atoms, fact_hashes)


# speedup vs baseline: 5.7798x; 1.0576x over previous
import functools

import jax
import jax.numpy as jnp
from jax import lax
from jax.experimental import pallas as pl
from jax.experimental.pallas import tpu as pltpu
from jax.experimental.pallas import tpu_sc as plsc

_PACK_BASE = 100003
_B_LO = 34467
_NC, _NS = 2, 16
_NW = _NC * _NS
_L = 16
_C = 128
_W = 64
_BLK = 8
_SGN = -0x80000000


def _u32(x):
    return plsc.bitcast(x, jnp.uint32)


def _i32(x):
    return plsc.bitcast(x, jnp.int32)


def _carry(x, y, s):
    return ((x & y) | ((x | y) & ~s)) >> 31


def _pack_limbs(a, b, c):
    a, b, c = _u32(a), _u32(b), _u32(c)
    t = a * _B_LO
    s = t + b
    a_sh = a << 16
    u_lo = a_sh + s
    u_hi = (a >> 16) + _carry(a_sh, s, u_lo)
    p1 = u_lo >> 16
    p0 = u_lo & 0xFFFF
    q = p1 * _B_LO + p0
    q_sh = q << 16
    r = p0 * _B_LO
    x1 = q_sh + r
    c2 = _carry(q_sh, r, x1)
    key_lo = x1 + c
    c3 = _carry(x1, c, key_lo)
    key_hi = u_hi * _PACK_BASE + p1 + (q >> 16) + c2 + c3
    return _i32(key_hi), _i32(key_lo) ^ _SGN


def _make_sc_search(n, rows, sample, steps_a, chunks):
    mesh = plsc.VectorSubcoreMesh(
        core_axis_name="c", subcore_axis_name="s",
        num_cores=_NC, num_subcores=_NS)

    @functools.partial(
        pl.kernel,
        out_type=jax.ShapeDtypeStruct((n,), jnp.int32),
        mesh=mesh,
        scratch_types=[
            pltpu.VMEM((2 * sample,), jnp.int32),
            pltpu.VMEM((2, _C), jnp.int32),
            pltpu.VMEM((2, _C), jnp.int32),
            pltpu.VMEM((2, _C), jnp.int32),
            pltpu.VMEM((2, _C), jnp.int32),
            pltpu.VMEM((2, _C), jnp.int32),
            pltpu.VMEM((2, _C), jnp.int32),
            pltpu.VMEM((2, _C, 2 * _W), jnp.int32),
            pltpu.VMEM((2, _C), jnp.int32),
            pltpu.SemaphoreType.DMA,
            pltpu.SemaphoreType.DMA,
            pltpu.SemaphoreType.DMA,
            pltpu.SemaphoreType.DMA,
            pltpu.SemaphoreType.DMA,
            pltpu.SemaphoreType.DMA,
        ],
        compiler_params=pltpu.CompilerParams(needs_layout_passes=False),
    )
    def sc_search(a_hbm, b_hbm, c_hbm, fcomb_hbm, smp_hbm, out_hbm,
                  smp, av, bv, cv, bktv, qhv, qlv, win, outv,
                  gsem0, gsem1, qsem0, qsem1, osem0, osem1):
        wid = lax.axis_index("s") * _NC + lax.axis_index("c")
        pltpu.sync_copy(smp_hbm, smp)
        tile_base = wid * jnp.int32(chunks * _C)
        gsems = (gsem0, gsem1)
        qsems = (qsem0, qsem1)
        osems = (osem0, osem1)
        ngrp = _C // _L
        last_base = jnp.int32(n - _C)

        def cbase(k):
            k = jnp.maximum(k, jnp.int32(0))
            return jnp.minimum(tile_base + k * jnp.int32(_C), last_base)

        def q_copies(base, buf):
            b = jnp.int32(buf)
            sl = pl.ds(base, _C)
            return [(a_hbm.at[sl], av.at[b]), (b_hbm.at[sl], bv.at[b]),
                    (c_hbm.at[sl], cv.at[b])]

        def qfire(base, buf):
            for src, dst in q_copies(base, buf):
                pltpu.async_copy(src, dst, qsems[buf])

        def qwait(base, buf):
            for src, dst in q_copies(base, buf):
                pltpu.make_async_copy(src, dst, qsems[buf]).wait()

        def phase_a(buf):
            buf = jnp.int32(buf)
            for blk in range(0, ngrp, _BLK):
                keys, pos = [], []
                for g in range(blk, blk + _BLK):
                    a16 = av[buf, pl.ds(g * _L, _L)]
                    b16 = bv[buf, pl.ds(g * _L, _L)]
                    c16 = cv[buf, pl.ds(g * _L, _L)]
                    qh, ql = _pack_limbs(a16, b16, c16)
                    qhv[buf, pl.ds(g * _L, _L)] = qh
                    qlv[buf, pl.ds(g * _L, _L)] = ql
                    keys.append((qh, ql))
                    pos.append(jnp.zeros((_L,), jnp.int32))
                for sstep in range(steps_a - 1, -1, -1):
                    ts, ths, tls = [], [], []
                    for i in range(_BLK):
                        t = pos[i] + (1 << sstep)
                        t2 = t * 2
                        ts.append(t)
                        ths.append(plsc.load_gather(smp, [t2 - 1]))
                        tls.append(plsc.load_gather(smp, [t2 - 2]) ^ _SGN)
                    for i in range(_BLK):
                        qh, ql = keys[i]
                        less = (ths[i] < qh) | ((ths[i] == qh) & (tls[i] < ql))
                        pos[i] = jnp.where(less, ts[i], pos[i])
                for i in range(_BLK):
                    bktv[buf, pl.ds((blk + i) * _L, _L)] = (
                        jnp.minimum(pos[i], rows - 1))

        def gfire(buf):
            pltpu.async_copy(
                fcomb_hbm.at[bktv.at[jnp.int32(buf)]],
                win.at[jnp.int32(buf)], gsems[buf])

        def gwait(buf):
            pltpu.make_async_copy(
                fcomb_hbm.at[bktv.at[jnp.int32(buf)]],
                win.at[jnp.int32(buf)], gsems[buf]).wait()

        def ofire(base, buf):
            pltpu.async_copy(
                outv.at[jnp.int32(buf)], out_hbm.at[pl.ds(base, _C)],
                osems[buf])

        def owait(base, buf):
            pltpu.make_async_copy(
                outv.at[jnp.int32(buf)], out_hbm.at[pl.ds(base, _C)],
                osems[buf]).wait()

        def phase_b(buf):
            buf = jnp.int32(buf)
            for blk in range(0, ngrp, _BLK):
                keys, pos, rws = [], [], []
                for g in range(blk, blk + _BLK):
                    qh = qhv[buf, pl.ds(g * _L, _L)]
                    ql = qlv[buf, pl.ds(g * _L, _L)]
                    keys.append((qh, ql))
                    rws.append(lax.iota(jnp.int32, _L) + (g * _L))
                    pos.append(jnp.zeros((_L,), jnp.int32))
                wref = win.at[buf]
                for sstep in range(5, -1, -1):
                    ts, ths, tls = [], [], []
                    for i in range(_BLK):
                        t = pos[i] + (1 << sstep)
                        ts.append(t)
                        ths.append(
                            plsc.load_gather(wref, [rws[i], t + (_W - 1)]))
                        tls.append(
                            plsc.load_gather(wref, [rws[i], t - 1]) ^ _SGN)
                    for i in range(_BLK):
                        qh, ql = keys[i]
                        less = (ths[i] < qh) | ((ths[i] == qh) & (tls[i] < ql))
                        pos[i] = jnp.where(less, ts[i], pos[i])
                for i in range(_BLK):
                    qh, ql = keys[i]
                    fh = plsc.load_gather(wref, [rws[i], pos[i] + _W])
                    fl = plsc.load_gather(wref, [rws[i], pos[i]]) ^ _SGN
                    hit = (fh == qh) & (fl == ql)
                    outv[buf, pl.ds((blk + i) * _L, _L)] = (
                        hit.astype(jnp.int32))

        pltpu.async_copy(out_hbm.at[pl.ds(cbase(jnp.int32(0)), _C)],
                         outv.at[jnp.int32(0)], osem0)
        pltpu.async_copy(out_hbm.at[pl.ds(cbase(jnp.int32(0)), _C)],
                         outv.at[jnp.int32(1)], osem1)
        qfire(cbase(jnp.int32(0)), 0)
        qwait(cbase(jnp.int32(0)), 0)
        qfire(cbase(jnp.int32(1)), 1)
        phase_a(0)
        gfire(0)

        def pair_body(j, carry):
            k = j * jnp.int32(2)
            qwait(cbase(k + 1), 1)
            qfire(cbase(k + 2), 0)
            phase_a(1)
            gfire(1)
            gwait(0)
            owait(cbase(k - 2), 0)
            phase_b(0)
            ofire(cbase(k), 0)

            qwait(cbase(k + 2), 0)
            qfire(cbase(k + 3), 1)
            phase_a(0)
            gfire(0)
            gwait(1)
            owait(cbase(k - 1), 1)
            phase_b(1)
            ofire(cbase(k + 1), 1)
            return carry

        nloop = jnp.int32((chunks - 1) // 2)
        lax.fori_loop(jnp.int32(0), nloop, pair_body, jnp.int32(0))
        lastk = jnp.int32(chunks - 1)
        qwait(cbase(lastk + 1), 1)
        gwait(0)
        owait(cbase(lastk - 2), 0)
        phase_b(0)
        ofire(cbase(lastk), 0)
        owait(cbase(lastk - 1), 1)
        owait(cbase(lastk), 0)

    return sc_search


@jax.jit
def _fact_index(atoms, fact_hashes):
    n = atoms.shape[0]
    f = fact_hashes.shape[0]

    if f % _W:
        pad = _W - f % _W
        fact_hashes = jnp.concatenate(
            [fact_hashes, jnp.full((pad,), (1 << 62), jnp.int64)])
        f += pad
    rows = f // _W
    fhi = (fact_hashes >> 32).astype(jnp.int32)
    flo = fact_hashes.astype(jnp.int32)
    fcomb = jnp.concatenate(
        [flo.reshape(rows, _W), fhi.reshape(rows, _W)], axis=1)
    sample = max(2, 1 << (rows - 1).bit_length())
    steps_a = sample.bit_length() - 1
    smp64 = jnp.max(fact_hashes.reshape(rows, _W), axis=1)
    smp_hi = jnp.concatenate(
        [(smp64 >> 32).astype(jnp.int32),
         jnp.full((sample - rows,), 0x7FFFFFFF, jnp.int32)])
    smp_lo = jnp.concatenate(
        [smp64.astype(jnp.int32), jnp.full((sample - rows,), -1, jnp.int32)])
    smp_flat = jnp.stack([smp_lo, smp_hi], axis=1).reshape(-1)

    cols = [atoms[:, j].astype(jnp.int32) for j in range(3)]
    chunks = (n + _NW * _C - 1) // (_NW * _C)
    if chunks % 2 == 0:
        chunks += 1

    sc_search = _make_sc_search(n, rows, sample, steps_a, chunks)
    out = sc_search(cols[0], cols[1], cols[2], fcomb, smp_flat)
    return out != 0


def kernel(atoms, fact_hashes):
    if atoms.shape[0] == 0 or fact_hashes.shape[0] == 0:
        return jnp.zeros((atoms.shape[0],), dtype=bool)
    return _fact_index(atoms, fact_hashes)
